# baseline (device time: 59113 ns/iter reference)
import jax
import jax.numpy as jnp
from jax import lax
from jax.experimental import pallas as pl
from jax.experimental.pallas import tpu as pltpu

N_DEV = 16
B, SQ, SKV = 2, 512, 512
H_LOC, DH = 8, 64
D_MODEL = 768
D_LOC = H_LOC * DH
CHUNK = SQ // N_DEV
HALF = SQ // 2

_X_OF_J = (0, 1, 1, 0)
_Y_OF_J = (0, 0, 1, 1)
_J_OF_XY = {(0, 0): 0, (1, 0): 1, (1, 1): 2, (0, 1): 3}


def _owner_of_chunk(c: int) -> int:
    x, y = c & 1, (c >> 1) & 1
    z = ((c >> 2) & 1) + 2 * ((c >> 3) & 1)
    return 4 * z + _J_OF_XY[(x, y)]


def _owner_row_start(rank: int) -> int:
    j, z = rank % 4, rank // 4
    c = _X_OF_J[j] + 2 * _Y_OF_J[j] + 4 * (z % 2) + 8 * (z // 2)
    return c * CHUNK


def kernel(x, Wq, K_ext, V_ext, Wo):
    def body(x_ref, wq_hbm, k_ref, v_ref, wo_hbm, out_ref,
             wq_v, wo_v, wo_bf, ctx_v, part_bf, a2a_buf, bc_stage, bc_buf,
             local_sems, a2a_send, a2a_recv, bc_send, bc_recv):
        bf = jnp.bfloat16
        my = lax.axis_index("i")
        j = my % 4
        z = my // 4
        mx = jnp.where((j == 1) | (j == 2), 1, 0)
        my_y = jnp.where(j >= 2, 1, 0)
        c_mine = mx + 2 * my_y + 4 * (z % 2) + 8 * (z // 2)
        my_rows = c_mine * CHUNK
        h_mine = c_mine // 8

        barrier = pltpu.get_barrier_semaphore()
        for k in range(1, N_DEV):
            pl.semaphore_signal(
                barrier, inc=1,
                device_id=((my + k) % N_DEV,),
                device_id_type=pl.DeviceIdType.MESH,
            )
        pl.semaphore_wait(barrier, N_DEV - 1)

        cp_wq = pltpu.make_async_copy(
            wq_hbm.at[:, pl.ds(my * D_LOC, D_LOC)], wq_v, local_sems.at[0])
        cp_wo = pltpu.make_async_copy(
            wo_hbm.at[pl.ds(my * D_LOC, D_LOC), :], wo_v, local_sems.at[1])
        cp_wq.start()
        cp_wo.start()
        cp_wq.wait()
        wq_v[...] = wq_v[...] * 0.125
        cp_wo.wait()
        wo_bf[...] = wo_v[...].astype(bf)

        rb = lax.broadcasted_iota(jnp.int32, (SQ, SKV), 0) // 64
        cb = lax.broadcasted_iota(jnp.int32, (SQ, SKV), 1) // 64
        mask = (rb == cb) | (cb == 0) | ((rb + cb) % 3 == 0)

        def compute_quarter(b, half):
            r0 = half * HALF
            mask_q = mask[r0:r0 + HALF, :]
            q_q = jnp.dot(x_ref[b, r0:r0 + HALF, :], wq_v[...],
                          preferred_element_type=jnp.float32)
            for h in range(H_LOC):
                q_h = q_q[:, h * DH:(h + 1) * DH]
                k_h = k_ref[b, :, h, :]
                s = lax.dot_general(
                    q_h, k_h, (((1,), (1,)), ((), ())),
                    preferred_element_type=jnp.float32)
                w = jnp.exp(jnp.where(mask_q, s, jnp.float32(-1e9)))
                rcp = 1.0 / jnp.sum(w, axis=1, keepdims=True)
                ctx_v[:, h * DH:(h + 1) * DH] = jnp.dot(
                    w, v_ref[b, :, h, :],
                    preferred_element_type=jnp.float32) * rcp
            part_bf[b, r0:r0 + HALF, :] = jnp.dot(
                ctx_v[...].astype(bf), wo_bf[...],
                preferred_element_type=jnp.float32).astype(bf)

        a2a = []

        def send_a2a(b, half):
            for c in range(half * 8, half * 8 + 8):
                owner = _owner_of_chunk(c)
                rdma = pltpu.make_async_remote_copy(
                    src_ref=part_bf.at[b, pl.ds(c * CHUNK, CHUNK), :],
                    dst_ref=a2a_buf.at[my, b],
                    send_sem=a2a_send.at[c * B + b],
                    recv_sem=a2a_recv.at[my * B + b],
                    device_id=(owner,),
                    device_id_type=pl.DeviceIdType.MESH,
                )
                a2a.append((c, rdma))

                @pl.when(my != owner)
                def _(rdma=rdma):
                    rdma.start()

        bc = []

        def reduce_and_bcast(b, cond=None):
            def gate(extra):
                return extra if cond is None else (cond & extra)

            for i in range(N_DEV):
                @pl.when(gate(my != i))
                def _(i=i):
                    pltpu.make_async_copy(
                        a2a_buf.at[i, b], a2a_buf.at[i, b],
                        a2a_recv.at[i * B + b],
                    ).wait()

            def do_reduce():
                acc = part_bf[b, pl.ds(my_rows, CHUNK), :].astype(
                    jnp.float32)
                for i in range(N_DEV):
                    acc = acc + jnp.where(my == i, jnp.float32(0.0),
                                          a2a_buf[i, b].astype(jnp.float32))
                out_ref[b, pl.ds(my_rows, CHUNK), :] = acc
                bc_stage[b, :, :] = acc.astype(bf)

            if cond is None:
                do_reduce()
            else:
                pl.when(cond)(do_reduce)

            for i in range(N_DEV):
                rdma = pltpu.make_async_remote_copy(
                    src_ref=bc_stage.at[b],
                    dst_ref=bc_buf.at[my, b],
                    send_sem=bc_send.at[i * B + b],
                    recv_sem=bc_recv.at[my * B + b],
                    device_id=(i,),
                    device_id_type=pl.DeviceIdType.MESH,
                )
                bc.append((i, cond, rdma))

                @pl.when(gate(my != i))
                def _(rdma=rdma):
                    rdma.start()

        compute_quarter(0, 0)
        send_a2a(0, 0)
        compute_quarter(0, 1)
        send_a2a(0, 1)

        reduce_and_bcast(0, cond=(h_mine == 0))
        compute_quarter(1, 0)
        send_a2a(1, 0)
        reduce_and_bcast(0, cond=(h_mine == 1))
        compute_quarter(1, 1)
        send_a2a(1, 1)
        reduce_and_bcast(1)

        for b in range(B):
            for i in range(N_DEV):
                rows_i = _owner_row_start(i)

                @pl.when(my != i)
                def _(i=i, b=b, rows_i=rows_i):
                    pltpu.make_async_copy(
                        bc_buf.at[i, b], bc_buf.at[i, b],
                        bc_recv.at[i * B + b],
                    ).wait()
                    out_ref[b, pl.ds(rows_i, CHUNK), :] = (
                        bc_buf[i, b].astype(jnp.float32))

        for c, rdma in a2a:
            @pl.when(my != _owner_of_chunk(c))
            def _(rdma=rdma):
                rdma.wait_send()
        for i, cond, rdma in bc:
            full = (my != i) if cond is None else ((my != i) & cond)

            @pl.when(full)
            def _(rdma=rdma):
                rdma.wait_send()

    return pl.pallas_call(
        body,
        out_shape=jax.ShapeDtypeStruct((B, SQ, D_MODEL), jnp.float32),
        in_specs=[
            pl.BlockSpec(memory_space=pltpu.VMEM),
            pl.BlockSpec(memory_space=pl.ANY),
            pl.BlockSpec(memory_space=pltpu.VMEM),
            pl.BlockSpec(memory_space=pltpu.VMEM),
            pl.BlockSpec(memory_space=pl.ANY),
        ],
        out_specs=pl.BlockSpec(memory_space=pltpu.VMEM),
        scratch_shapes=[
            pltpu.VMEM((D_MODEL, D_LOC), jnp.float32),
            pltpu.VMEM((D_LOC, D_MODEL), jnp.float32),
            pltpu.VMEM((D_LOC, D_MODEL), jnp.bfloat16),
            pltpu.VMEM((HALF, D_LOC), jnp.float32),
            pltpu.VMEM((B, SQ, D_MODEL), jnp.bfloat16),
            pltpu.VMEM((N_DEV, B, CHUNK, D_MODEL), jnp.bfloat16),
            pltpu.VMEM((B, CHUNK, D_MODEL), jnp.bfloat16),
            pltpu.VMEM((N_DEV, B, CHUNK, D_MODEL), jnp.bfloat16),
            pltpu.SemaphoreType.DMA((2,)),
            pltpu.SemaphoreType.DMA((N_DEV * B,)),
            pltpu.SemaphoreType.DMA((N_DEV * B,)),
            pltpu.SemaphoreType.DMA((N_DEV * B,)),
            pltpu.SemaphoreType.DMA((N_DEV * B,)),
        ],
        compiler_params=pltpu.CompilerParams(collective_id=0),
    )(x, Wq, K_ext, V_ext, Wo)


# device time: 57041 ns/iter; 1.0363x vs baseline; 1.0363x over previous
import jax
import jax.numpy as jnp
from jax import lax
from jax.experimental import pallas as pl
from jax.experimental.pallas import tpu as pltpu

N_DEV = 16
B, SQ, SKV = 2, 512, 512
H_LOC, DH = 8, 64
D_MODEL = 768
D_LOC = H_LOC * DH
CHUNK = SQ // N_DEV

_X_OF_J = (0, 1, 1, 0)
_Y_OF_J = (0, 0, 1, 1)
_J_OF_XY = {(0, 0): 0, (1, 0): 1, (1, 1): 2, (0, 1): 3}


def _owner_of_chunk(c: int) -> int:
    x, y = c & 1, (c >> 1) & 1
    z = ((c >> 2) & 1) + 2 * ((c >> 3) & 1)
    return 4 * z + _J_OF_XY[(x, y)]


def _owner_row_start(rank: int) -> int:
    j, z = rank % 4, rank // 4
    c = _X_OF_J[j] + 2 * _Y_OF_J[j] + 4 * (z % 2) + 8 * (z // 2)
    return c * CHUNK


def kernel(x, Wq, K_ext, V_ext, Wo):
    def body(x_ref, wq_hbm, k_ref, v_ref, wo_hbm, out_ref,
             wq_v, wq_bf, wo_v, wo_bf, ctx_v, part_bf, a2a_buf, bc_stage,
             bc_buf, local_sems, a2a_send, a2a_recv, bc_send, bc_recv):
        bf = jnp.bfloat16
        my = lax.axis_index("i")
        j = my % 4
        z = my // 4
        mx = jnp.where((j == 1) | (j == 2), 1, 0)
        my_y = jnp.where(j >= 2, 1, 0)
        c_mine = mx + 2 * my_y + 4 * (z % 2) + 8 * (z // 2)
        my_rows = c_mine * CHUNK

        cp_wq = pltpu.make_async_copy(
            wq_hbm.at[:, pl.ds(my * D_LOC, D_LOC)], wq_v, local_sems.at[0])
        cp_wo = pltpu.make_async_copy(
            wo_hbm.at[pl.ds(my * D_LOC, D_LOC), :], wo_v, local_sems.at[1])
        cp_wq.start()
        cp_wo.start()
        cp_wq.wait()
        wq_bf[...] = (wq_v[...] * 0.125).astype(bf)
        cp_wo.wait()
        wo_bf[...] = wo_v[...].astype(bf)

        rb = lax.broadcasted_iota(jnp.int32, (SQ, SKV), 0) // 64
        cb = lax.broadcasted_iota(jnp.int32, (SQ, SKV), 1) // 64
        mask = (rb == cb) | (cb == 0) | ((rb + cb) % 3 == 0)

        a2a = []

        def compute_batch(b):
            q_b = jnp.dot(x_ref[b].astype(bf), wq_bf[...],
                          preferred_element_type=jnp.float32)
            q_bb = q_b.astype(bf)
            for h in range(H_LOC):
                q_h = q_bb[:, h * DH:(h + 1) * DH]
                k_h = k_ref[b, :, h, :].astype(bf)
                s = lax.dot_general(
                    q_h, k_h, (((1,), (1,)), ((), ())),
                    preferred_element_type=jnp.float32)
                w = jnp.exp(jnp.where(mask, s, jnp.float32(-1e9)))
                rcp = 1.0 / jnp.sum(w, axis=1, keepdims=True)
                ctx_v[:, h * DH:(h + 1) * DH] = jnp.dot(
                    w, v_ref[b, :, h, :],
                    preferred_element_type=jnp.float32) * rcp
            part_bf[b, :, :] = jnp.dot(
                ctx_v[...].astype(bf), wo_bf[...],
                preferred_element_type=jnp.float32).astype(bf)

        def send_a2a(b):
            for c in range(N_DEV):
                owner = _owner_of_chunk(c)
                rdma = pltpu.make_async_remote_copy(
                    src_ref=part_bf.at[b, pl.ds(c * CHUNK, CHUNK), :],
                    dst_ref=a2a_buf.at[my, b],
                    send_sem=a2a_send.at[c * B + b],
                    recv_sem=a2a_recv.at[my * B + b],
                    device_id=(owner,),
                    device_id_type=pl.DeviceIdType.MESH,
                )
                a2a.append((c, rdma))

                @pl.when(my != owner)
                def _(rdma=rdma):
                    rdma.start()

        bc = []

        def reduce_and_bcast(b):
            for i in range(N_DEV):
                @pl.when(my != i)
                def _(i=i):
                    pltpu.make_async_copy(
                        a2a_buf.at[i, b], a2a_buf.at[i, b],
                        a2a_recv.at[i * B + b],
                    ).wait()

            acc = part_bf[b, pl.ds(my_rows, CHUNK), :].astype(jnp.float32)
            for i in range(N_DEV):
                acc = acc + jnp.where(my == i, jnp.float32(0.0),
                                      a2a_buf[i, b].astype(jnp.float32))
            out_ref[b, pl.ds(my_rows, CHUNK), :] = acc
            bc_stage[b, :, :] = acc.astype(bf)

            for i in range(N_DEV):
                rdma = pltpu.make_async_remote_copy(
                    src_ref=bc_stage.at[b],
                    dst_ref=bc_buf.at[my, b],
                    send_sem=bc_send.at[i * B + b],
                    recv_sem=bc_recv.at[my * B + b],
                    device_id=(i,),
                    device_id_type=pl.DeviceIdType.MESH,
                )
                bc.append((i, rdma))

                @pl.when(my != i)
                def _(rdma=rdma):
                    rdma.start()

        compute_batch(0)
        barrier = pltpu.get_barrier_semaphore()
        for k in range(1, N_DEV):
            pl.semaphore_signal(
                barrier, inc=1,
                device_id=((my + k) % N_DEV,),
                device_id_type=pl.DeviceIdType.MESH,
            )
        pl.semaphore_wait(barrier, N_DEV - 1)
        send_a2a(0)
        compute_batch(1)
        send_a2a(1)
        reduce_and_bcast(0)
        reduce_and_bcast(1)

        for b in range(B):
            for i in range(N_DEV):
                rows_i = _owner_row_start(i)

                @pl.when(my != i)
                def _(i=i, b=b, rows_i=rows_i):
                    pltpu.make_async_copy(
                        bc_buf.at[i, b], bc_buf.at[i, b],
                        bc_recv.at[i * B + b],
                    ).wait()
                    out_ref[b, pl.ds(rows_i, CHUNK), :] = (
                        bc_buf[i, b].astype(jnp.float32))

        for c, rdma in a2a:
            @pl.when(my != _owner_of_chunk(c))
            def _(rdma=rdma):
                rdma.wait_send()
        for i, rdma in bc:
            @pl.when(my != i)
            def _(rdma=rdma):
                rdma.wait_send()

    return pl.pallas_call(
        body,
        out_shape=jax.ShapeDtypeStruct((B, SQ, D_MODEL), jnp.float32),
        in_specs=[
            pl.BlockSpec(memory_space=pltpu.VMEM),
            pl.BlockSpec(memory_space=pl.ANY),
            pl.BlockSpec(memory_space=pltpu.VMEM),
            pl.BlockSpec(memory_space=pltpu.VMEM),
            pl.BlockSpec(memory_space=pl.ANY),
        ],
        out_specs=pl.BlockSpec(memory_space=pltpu.VMEM),
        scratch_shapes=[
            pltpu.VMEM((D_MODEL, D_LOC), jnp.float32),
            pltpu.VMEM((D_MODEL, D_LOC), jnp.bfloat16),
            pltpu.VMEM((D_LOC, D_MODEL), jnp.float32),
            pltpu.VMEM((D_LOC, D_MODEL), jnp.bfloat16),
            pltpu.VMEM((SQ, D_LOC), jnp.float32),
            pltpu.VMEM((B, SQ, D_MODEL), jnp.bfloat16),
            pltpu.VMEM((N_DEV, B, CHUNK, D_MODEL), jnp.bfloat16),
            pltpu.VMEM((B, CHUNK, D_MODEL), jnp.bfloat16),
            pltpu.VMEM((N_DEV, B, CHUNK, D_MODEL), jnp.bfloat16),
            pltpu.SemaphoreType.DMA((2,)),
            pltpu.SemaphoreType.DMA((N_DEV * B,)),
            pltpu.SemaphoreType.DMA((N_DEV * B,)),
            pltpu.SemaphoreType.DMA((N_DEV * B,)),
            pltpu.SemaphoreType.DMA((N_DEV * B,)),
        ],
        compiler_params=pltpu.CompilerParams(collective_id=0),
    )(x, Wq, K_ext, V_ext, Wo)


# device time: 56093 ns/iter; 1.0538x vs baseline; 1.0169x over previous
import jax
import jax.numpy as jnp
from jax import lax
from jax.experimental import pallas as pl
from jax.experimental.pallas import tpu as pltpu

N_DEV = 16
B, SQ, SKV = 2, 512, 512
H_LOC, DH = 8, 64
D_MODEL = 768
D_LOC = H_LOC * DH
CHUNK = SQ // N_DEV

_X_OF_J = (0, 1, 1, 0)
_Y_OF_J = (0, 0, 1, 1)
_J_OF_XY = {(0, 0): 0, (1, 0): 1, (1, 1): 2, (0, 1): 3}


def _owner_of_chunk(c: int) -> int:
    x, y = c & 1, (c >> 1) & 1
    z = ((c >> 2) & 1) + 2 * ((c >> 3) & 1)
    return 4 * z + _J_OF_XY[(x, y)]


def _owner_row_start(rank: int) -> int:
    j, z = rank % 4, rank // 4
    c = _X_OF_J[j] + 2 * _Y_OF_J[j] + 4 * (z % 2) + 8 * (z // 2)
    return c * CHUNK


def kernel(x, Wq, K_ext, V_ext, Wo):
    def body(x_ref, wq_hbm, k_ref, v_ref, wo_hbm, out_ref,
             wq_v, wo_v, wo_bf, ctx_v, part_bf, a2a_buf, bc_stage, bc_buf,
             local_sems, a2a_send, a2a_recv, bc_send, bc_recv):
        bf = jnp.bfloat16
        my = lax.axis_index("i")
        j = my % 4
        z = my // 4
        mx = jnp.where((j == 1) | (j == 2), 1, 0)
        my_y = jnp.where(j >= 2, 1, 0)
        c_mine = mx + 2 * my_y + 4 * (z % 2) + 8 * (z // 2)
        my_rows = c_mine * CHUNK

        barrier = pltpu.get_barrier_semaphore()
        for k in range(1, N_DEV):
            pl.semaphore_signal(
                barrier, inc=1,
                device_id=((my + k) % N_DEV,),
                device_id_type=pl.DeviceIdType.MESH,
            )
        pl.semaphore_wait(barrier, N_DEV - 1)

        cp_wq = pltpu.make_async_copy(
            wq_hbm.at[:, pl.ds(my * D_LOC, D_LOC)], wq_v, local_sems.at[0])
        cp_wo = pltpu.make_async_copy(
            wo_hbm.at[pl.ds(my * D_LOC, D_LOC), :], wo_v, local_sems.at[1])
        cp_wq.start()
        cp_wo.start()
        cp_wq.wait()
        wq_v[...] = wq_v[...] * 0.125
        cp_wo.wait()
        wo_bf[...] = wo_v[...].astype(bf)

        rb = lax.broadcasted_iota(jnp.int32, (SQ, SKV), 0) // 64
        cb = lax.broadcasted_iota(jnp.int32, (SQ, SKV), 1) // 64
        mask = (rb == cb) | (cb == 0) | ((rb + cb) % 3 == 0)

        a2a = []

        def compute_batch(b):
            q_b = jnp.dot(x_ref[b], wq_v[...],
                          preferred_element_type=jnp.float32)
            for h in range(H_LOC):
                q_h = q_b[:, h * DH:(h + 1) * DH]
                k_h = k_ref[b, :, h, :]
                s = lax.dot_general(
                    q_h, k_h, (((1,), (1,)), ((), ())),
                    preferred_element_type=jnp.float32)
                w = jnp.exp(jnp.where(mask, s, jnp.float32(-1e9)))
                rcp = 1.0 / jnp.sum(w, axis=1, keepdims=True)
                ctx_v[:, h * DH:(h + 1) * DH] = jnp.dot(
                    w, v_ref[b, :, h, :],
                    preferred_element_type=jnp.float32) * rcp
            part_bf[b, :, :] = jnp.dot(
                ctx_v[...].astype(bf), wo_bf[...],
                preferred_element_type=jnp.float32).astype(bf)

        def send_a2a(b):
            for c in range(N_DEV):
                owner = _owner_of_chunk(c)
                rdma = pltpu.make_async_remote_copy(
                    src_ref=part_bf.at[b, pl.ds(c * CHUNK, CHUNK), :],
                    dst_ref=a2a_buf.at[my, b],
                    send_sem=a2a_send.at[c * B + b],
                    recv_sem=a2a_recv.at[my * B + b],
                    device_id=(owner,),
                    device_id_type=pl.DeviceIdType.MESH,
                )
                a2a.append((c, rdma))

                @pl.when(my != owner)
                def _(rdma=rdma):
                    rdma.start()

        bc = []

        def reduce_and_bcast(b):
            for i in range(N_DEV):
                @pl.when(my != i)
                def _(i=i):
                    pltpu.make_async_copy(
                        a2a_buf.at[i, b], a2a_buf.at[i, b],
                        a2a_recv.at[i * B + b],
                    ).wait()

            acc = part_bf[b, pl.ds(my_rows, CHUNK), :].astype(jnp.float32)
            for i in range(N_DEV):
                acc = acc + jnp.where(my == i, jnp.float32(0.0),
                                      a2a_buf[i, b].astype(jnp.float32))
            out_ref[b, pl.ds(my_rows, CHUNK), :] = acc
            bc_stage[b, :, :] = acc.astype(bf)

            for i in range(N_DEV):
                rdma = pltpu.make_async_remote_copy(
                    src_ref=bc_stage.at[b],
                    dst_ref=bc_buf.at[my, b],
                    send_sem=bc_send.at[i * B + b],
                    recv_sem=bc_recv.at[my * B + b],
                    device_id=(i,),
                    device_id_type=pl.DeviceIdType.MESH,
                )
                bc.append((i, rdma))

                @pl.when(my != i)
                def _(rdma=rdma):
                    rdma.start()

        compute_batch(0)
        send_a2a(0)
        compute_batch(1)
        send_a2a(1)
        reduce_and_bcast(0)
        reduce_and_bcast(1)

        for b in range(B):
            for i in range(N_DEV):
                rows_i = _owner_row_start(i)

                @pl.when(my != i)
                def _(i=i, b=b, rows_i=rows_i):
                    pltpu.make_async_copy(
                        bc_buf.at[i, b], bc_buf.at[i, b],
                        bc_recv.at[i * B + b],
                    ).wait()
                    out_ref[b, pl.ds(rows_i, CHUNK), :] = (
                        bc_buf[i, b].astype(jnp.float32))

        for c, rdma in a2a:
            @pl.when(my != _owner_of_chunk(c))
            def _(rdma=rdma):
                rdma.wait_send()
        for i, rdma in bc:
            @pl.when(my != i)
            def _(rdma=rdma):
                rdma.wait_send()

    return pl.pallas_call(
        body,
        out_shape=jax.ShapeDtypeStruct((B, SQ, D_MODEL), jnp.float32),
        in_specs=[
            pl.BlockSpec(memory_space=pltpu.VMEM),
            pl.BlockSpec(memory_space=pl.ANY),
            pl.BlockSpec(memory_space=pltpu.VMEM),
            pl.BlockSpec(memory_space=pltpu.VMEM),
            pl.BlockSpec(memory_space=pl.ANY),
        ],
        out_specs=pl.BlockSpec(memory_space=pltpu.VMEM),
        scratch_shapes=[
            pltpu.VMEM((D_MODEL, D_LOC), jnp.float32),
            pltpu.VMEM((D_LOC, D_MODEL), jnp.float32),
            pltpu.VMEM((D_LOC, D_MODEL), jnp.bfloat16),
            pltpu.VMEM((SQ, D_LOC), jnp.float32),
            pltpu.VMEM((B, SQ, D_MODEL), jnp.bfloat16),
            pltpu.VMEM((N_DEV, B, CHUNK, D_MODEL), jnp.bfloat16),
            pltpu.VMEM((B, CHUNK, D_MODEL), jnp.bfloat16),
            pltpu.VMEM((N_DEV, B, CHUNK, D_MODEL), jnp.bfloat16),
            pltpu.SemaphoreType.DMA((2,)),
            pltpu.SemaphoreType.DMA((N_DEV * B,)),
            pltpu.SemaphoreType.DMA((N_DEV * B,)),
            pltpu.SemaphoreType.DMA((N_DEV * B,)),
            pltpu.SemaphoreType.DMA((N_DEV * B,)),
        ],
        compiler_params=pltpu.CompilerParams(collective_id=0),
    )(x, Wq, K_ext, V_ext, Wo)
